# Initial kernel scaffold; baseline (speedup 1.0000x reference)
import jax
import jax.numpy as jnp
from jax.experimental import pallas as pl


def kernel(logical_attr, logical_edge_index, hw_attr, hw_edge_index, emb_matrix, W1l, b1l, W2l, b2l, W3l, b3l, W1h, b1h, W2h, b2h, W3h, b3h, lin_W, lin_b, lin2_W, lin2_b):
    raise NotImplementedError("write your pallas kernel here")


# speedup vs baseline: 89.6481x; 89.6481x over previous
import functools

import jax
import jax.numpy as jnp
from jax import lax
from jax.experimental import pallas as pl
from jax.experimental.pallas import tpu as pltpu
from jax.experimental.pallas import tpu_sc as plsc

L = 16
NS = 16

N_L, N_LP = 10000, 10240
N_H = 2048
E_L, E_H = 320000, 32768
HID = 64


def _rsqrt_vec(x):
    xi = plsc.bitcast(x, jnp.int32)
    y = plsc.bitcast(jnp.int32(0x5F3759DF) - (xi >> 1), jnp.float32)
    for _ in range(3):
        y = y * (1.5 - 0.5 * x * y * y)
    return y


def _make_backprop(n_pad, n_edges):
    n_per = n_pad // NS
    e_per = n_edges // NS
    f32 = jnp.float32
    mesh = plsc.VectorSubcoreMesh(
        core_axis_name="c", subcore_axis_name="s", num_cores=1)

    def body(src_hbm, dst_hbm, v_hbm, u1_hbm, u2_hbm, u3_hbm,
             src_v, dst_v, t_v, g_v, dl_v, slc_v, tn_v, red_v, stage_s, bcast_s):
        sid = lax.axis_index("s")
        ebase = sid * e_per
        nbase = sid * n_per
        ones = jnp.full((L,), 1.0, f32)

        pltpu.sync_copy(src_hbm.at[pl.ds(ebase, e_per)], src_v)
        pltpu.sync_copy(dst_hbm.at[pl.ds(ebase, e_per)], dst_v)
        pltpu.sync_copy(v_hbm, t_v)

        def zero_g(j, c):
            g_v[pl.ds(j * L, L)] = jnp.zeros((L,), f32)
            return c

        def scatter_ones(i, c):
            d16 = dst_v[pl.ds(i * L, L)]
            plsc.addupdate_scatter(g_v, [d16], ones)
            return c

        lax.fori_loop(0, n_pad // L, zero_g, 0)
        lax.fori_loop(0, e_per // L, scatter_ones, 0)

        pltpu.sync_copy(g_v, stage_s.at[sid])
        plsc.subcore_barrier()
        pltpu.sync_copy(stage_s.at[:, pl.ds(nbase, n_per)], red_v)

        def dl_body(j, c):
            a = red_v[0, pl.ds(j * L, L)]
            for t in range(1, NS):
                a = a + red_v[t, pl.ds(j * L, L)]
            slc_v[pl.ds(j * L, L)] = _rsqrt_vec(a + 1.0)
            return c

        lax.fori_loop(0, n_per // L, dl_body, 0)
        pltpu.sync_copy(slc_v, bcast_s.at[pl.ds(nbase, n_per)])
        plsc.subcore_barrier()
        pltpu.sync_copy(bcast_s, dl_v)

        def t0_body(j, c):
            sl = pl.ds(j * L, L)
            t_v[sl] = t_v[sl] * dl_v[sl]
            return c

        lax.fori_loop(0, n_pad // L, t0_body, 0)

        for u_hbm in (u1_hbm, u2_hbm, u3_hbm):
            lax.fori_loop(0, n_pad // L, zero_g, 0)

            def spmv_body(i, c):
                s16 = src_v[pl.ds(i * L, L)]
                d16 = dst_v[pl.ds(i * L, L)]
                td = plsc.load_gather(t_v, [d16])
                plsc.addupdate_scatter(g_v, [s16], td)
                return c

            lax.fori_loop(0, e_per // L, spmv_body, 0)

            pltpu.sync_copy(g_v, stage_s.at[sid])
            plsc.subcore_barrier()
            pltpu.sync_copy(stage_s.at[:, pl.ds(nbase, n_per)], red_v)

            def u_body(j, c):
                a = red_v[0, pl.ds(j * L, L)]
                for t in range(1, NS):
                    a = a + red_v[t, pl.ds(j * L, L)]
                gsl = pl.ds(nbase + j * L, L)
                u = dl_v[gsl] * (a + t_v[gsl])
                slc_v[pl.ds(j * L, L)] = u
                tn_v[pl.ds(j * L, L)] = dl_v[gsl] * u
                return c

            lax.fori_loop(0, n_per // L, u_body, 0)
            pltpu.sync_copy(slc_v, u_hbm.at[pl.ds(nbase, n_per)])
            pltpu.sync_copy(tn_v, bcast_s.at[pl.ds(nbase, n_per)])
            plsc.subcore_barrier()
            pltpu.sync_copy(bcast_s, t_v)

    return pl.kernel(
        body,
        out_type=(jax.ShapeDtypeStruct((n_pad,), f32),) * 3,
        mesh=mesh,
        compiler_params=pltpu.CompilerParams(needs_layout_passes=False),
        scratch_types=[
            pltpu.VMEM((e_per,), jnp.int32),
            pltpu.VMEM((e_per,), jnp.int32),
            pltpu.VMEM((n_pad,), f32),
            pltpu.VMEM((n_pad,), f32),
            pltpu.VMEM((n_pad,), f32),
            pltpu.VMEM((n_per,), f32),
            pltpu.VMEM((n_per,), f32),
            pltpu.VMEM((NS, n_per), f32),
            pltpu.VMEM_SHARED((NS, n_pad), f32),
            pltpu.VMEM_SHARED((n_pad,), f32),
        ],
    )


_bp_logical = _make_backprop(N_LP, E_L)
_bp_hw = _make_backprop(N_H, E_H)


_MV_RB = 1000


def _mv_body(v_ref, e_ref, o_ref):
    @pl.when(pl.program_id(0) == 0)
    def _():
        o_ref[...] = jnp.zeros_like(o_ref)
    o_ref[...] += lax.dot_general(
        v_ref[...], e_ref[...], (((0,), (0,)), ((), ())),
        preferred_element_type=jnp.float32,
        precision=lax.Precision.HIGHEST)


def _matvec(v2, emb):
    return pl.pallas_call(
        _mv_body,
        grid=(N_L // _MV_RB,),
        in_specs=[pl.BlockSpec((_MV_RB, 1), lambda i: (i, 0)),
                  pl.BlockSpec((_MV_RB, N_H), lambda i: (i, 0))],
        out_specs=pl.BlockSpec((1, N_H), lambda i: (0, 0)),
        out_shape=jax.ShapeDtypeStruct((1, N_H), jnp.float32),
    )(v2, emb)


def _final_body(v_ref, u1_ref, u2_ref, u3_ref, xl_ref,
                w_ref, z1_ref, z2_ref, z3_ref, xh_ref,
                w1l_ref, w2l_ref, w3l_ref, b1l_ref, b2l_ref, b3l_ref,
                w1h_ref, w2h_ref, w3h_ref, b1h_ref, b2h_ref, b3h_ref,
                linw_ref, linb_ref, lin2b_ref, o_ref):
    def dot(a, b):
        return lax.dot_general(a, b, (((1,), (0,)), ((), ())),
                               preferred_element_type=jnp.float32,
                               precision=lax.Precision.HIGHEST)

    S0 = jnp.sum(v_ref[...])
    S1 = jnp.sum(u1_ref[...])
    S2 = jnp.sum(u2_ref[...])
    T0 = jnp.sum(w_ref[...])
    T1 = jnp.sum(z1_ref[...])
    T2 = jnp.sum(z2_ref[...])
    p = dot(u3_ref[...], xl_ref[...])
    q = dot(z3_ref[...], xh_ref[...])
    w23l = dot(w2l_ref[...], w3l_ref[...])
    cl = (dot(dot(p, w1l_ref[...]), w23l)
          + S2 * dot(b1l_ref[...], w23l)
          + S1 * dot(b2l_ref[...], w3l_ref[...])
          + S0 * b3l_ref[...])
    w23h = dot(w2h_ref[...], w3h_ref[...])
    ch = (dot(dot(q, w1h_ref[...]), w23h)
          + T2 * dot(b1h_ref[...], w23h)
          + T1 * dot(b2h_ref[...], w3h_ref[...])
          + T0 * b3h_ref[...])
    y = (dot(cl, linw_ref[:HID, :]) + dot(ch, linw_ref[HID:, :])
         + S0 * linb_ref[...] + lin2b_ref[...])
    o_ref[...] = y


def _final(*args):
    return pl.pallas_call(
        _final_body,
        out_shape=jax.ShapeDtypeStruct((1, 1), jnp.float32),
    )(*args)


def kernel(logical_attr, logical_edge_index, hw_attr, hw_edge_index, emb_matrix,
           W1l, b1l, W2l, b2l, W3l, b3l, W1h, b1h, W2h, b2h, W3h, b3h,
           lin_W, lin_b, lin2_W, lin2_b):
    f32 = jnp.float32
    ei_l = logical_edge_index.astype(jnp.int32)
    ei_h = hw_edge_index.astype(jnp.int32)
    v = lin2_W[:, 0].astype(f32)
    v_pad = jnp.pad(v, (0, N_LP - N_L))

    u1, u2, u3 = _bp_logical(ei_l[0], ei_l[1], v_pad)
    w = _matvec(lin2_W.astype(f32), emb_matrix)
    z1, z2, z3 = _bp_hw(ei_h[0], ei_h[1], w[0])

    row = lambda x: x[:N_L].reshape(1, -1)
    y = _final(v.reshape(1, -1), row(u1), row(u2), row(u3), logical_attr,
               w, z1.reshape(1, -1), z2.reshape(1, -1), z3.reshape(1, -1), hw_attr,
               W1l, W2l, W3l, b1l.reshape(1, -1), b2l.reshape(1, -1), b3l.reshape(1, -1),
               W1h, W2h, W3h, b1h.reshape(1, -1), b2h.reshape(1, -1), b3h.reshape(1, -1),
               lin_W, lin_b.reshape(1, 1), lin2_b.reshape(1, 1))
    return y.reshape(1, 1, 1)
